# baseline (speedup 1.0000x reference)
import jax
import jax.numpy as jnp
from jax import lax
from jax.experimental import pallas as pl
from jax.experimental.pallas import tpu as pltpu

_TOK_BLK = 4096
_NBUF = 3


def _router_body(h_hbm, w_ref, b_ref, logits_hbm, idx_hbm,
                 hbuf, lo_st, ix_st, sem_h, sem_lo, sem_ix):
    n = h_hbm.shape[0]
    steps = n // _TOK_BLK
    w = w_ref[...]
    bias = b_ref[...]

    def h_copy(step, slot):
        return pltpu.make_async_copy(
            h_hbm.at[pl.ds(step * _TOK_BLK, _TOK_BLK), :],
            hbuf.at[slot], sem_h.at[slot])

    h_copy(0, 0).start()
    h_copy(1, 1).start()

    def body(i, carry):
        slot = lax.rem(i, _NBUF)
        stage = lax.rem(i, 2)

        @pl.when(i + 2 < steps)
        def _():
            h_copy(i + 2, lax.rem(i + 2, _NBUF)).start()

        h_copy(i, slot).wait()

        @pl.when(i >= 2)
        def _():
            pltpu.make_async_copy(
                lo_st.at[stage],
                logits_hbm.at[pl.ds((i - 2) * _TOK_BLK, _TOK_BLK), :],
                sem_lo.at[stage]).wait()
            pltpu.make_async_copy(
                ix_st.at[stage],
                idx_hbm.at[pl.ds((i - 2) * _TOK_BLK, _TOK_BLK)],
                sem_ix.at[stage]).wait()

        logits = lax.dot_general(hbuf[slot], w, (((1,), (1,)), ((), ())),
                                 preferred_element_type=jnp.float32)
        logits = logits + bias
        lo_st[stage] = logits
        lt = logits.T
        colmax = jnp.max(lt, axis=0, keepdims=True)
        eidx = lax.broadcasted_iota(jnp.int32, lt.shape, 0)
        masked = jnp.where(lt == colmax, eidx, lt.shape[0])
        ix_st[stage] = jnp.min(masked, axis=0)

        pltpu.make_async_copy(
            lo_st.at[stage],
            logits_hbm.at[pl.ds(i * _TOK_BLK, _TOK_BLK), :],
            sem_lo.at[stage]).start()
        pltpu.make_async_copy(
            ix_st.at[stage],
            idx_hbm.at[pl.ds(i * _TOK_BLK, _TOK_BLK)],
            sem_ix.at[stage]).start()
        return carry

    lax.fori_loop(0, steps, body, 0)

    for back in (2, 1):
        i = steps - back
        stage = i % 2
        pltpu.make_async_copy(
            lo_st.at[stage],
            logits_hbm.at[pl.ds(i * _TOK_BLK, _TOK_BLK), :],
            sem_lo.at[stage]).wait()
        pltpu.make_async_copy(
            ix_st.at[stage],
            idx_hbm.at[pl.ds(i * _TOK_BLK, _TOK_BLK)],
            sem_ix.at[stage]).wait()


def kernel(h, W, b):
    n, d = h.shape
    e = W.shape[0]
    logits, idx = pl.pallas_call(
        _router_body,
        in_specs=[
            pl.BlockSpec(memory_space=pl.ANY),
            pl.BlockSpec(memory_space=pltpu.VMEM),
            pl.BlockSpec(memory_space=pltpu.VMEM),
        ],
        out_specs=[
            pl.BlockSpec(memory_space=pl.ANY),
            pl.BlockSpec(memory_space=pl.ANY),
        ],
        out_shape=[
            jax.ShapeDtypeStruct((n, e), jnp.float32),
            jax.ShapeDtypeStruct((n,), jnp.int32),
        ],
        scratch_shapes=[
            pltpu.VMEM((_NBUF, _TOK_BLK, d), jnp.float32),
            pltpu.VMEM((2, _TOK_BLK, e), jnp.float32),
            pltpu.VMEM((2, _TOK_BLK), jnp.int32),
            pltpu.SemaphoreType.DMA((_NBUF,)),
            pltpu.SemaphoreType.DMA((2,)),
            pltpu.SemaphoreType.DMA((2,)),
        ],
    )(h, W, b.reshape(1, e))
    return (logits, idx)
